# baseline (speedup 1.0000x reference)
import jax
import jax.numpy as jnp
from jax import lax
from jax.experimental import pallas as pl
from jax.experimental.pallas import tpu as pltpu
from jax.experimental.pallas import tpu_sc as plsc

N = 10000
E = 320000
V = 100000
D = 128
H = 128
G = 64

NC = 2
NS = 16
NW = NC * NS

NP = 10240
CE = 80
KE = NP // NW // CE

NHALF = N // 2
TRR = 8
ACCR = NHALF + TRR
RPA = 312
RPAL = ACCR - (NS - 1) * RPA
HT = 624
HTL = N - (NS - 1) * HT

CA = 64
ET = 20480
WIN = 8
EP = ET * NS
LCAP = 832
BCAP = 16 * LCAP
KB = BCAP // CA

_mesh = plsc.VectorSubcoreMesh(
    core_axis_name="c", subcore_axis_name="s", num_cores=NC, num_subcores=NS)


def _embed_body(emb_hbm, xr_hbm, out_hbm, idx_v, buf_v, sem):
    cid = lax.axis_index("c")
    sid = lax.axis_index("s")
    wid = cid * NS + sid
    pltpu.sync_copy(xr_hbm.at[cid, sid], idx_v)
    base = wid * (KE * CE)

    def body(j, carry):
        pltpu.async_copy(emb_hbm.at[idx_v.at[j]], buf_v, sem).wait()
        pltpu.sync_copy(buf_v, out_hbm.at[pl.ds(base + j * CE, CE)])
        return carry

    lax.fori_loop(0, KE, body, 0)


@jax.jit
def _embed(emb, xr):
    return pl.kernel(
        _embed_body,
        out_type=jax.ShapeDtypeStruct((NP, D), jnp.float32),
        mesh=_mesh,
        scratch_types=[
            pltpu.VMEM((KE, CE), jnp.int32),
            pltpu.VMEM((CE, D), jnp.float32),
            pltpu.SemaphoreType.DMA,
        ],
    )(emb, xr)


def _part_body(srcf_hbm, dstf_hbm, sb_hbm, db_hbm, src_v, dst_v,
               sb0_v, sb1_v, db0_v, db1_v):
    cid = lax.axis_index("c")
    sid = lax.axis_index("s")

    pltpu.sync_copy(srcf_hbm.at[sid], src_v)
    pltpu.sync_copy(dstf_hbm.at[sid], dst_v)

    zs = jnp.zeros((16,), jnp.int32)
    ns = jnp.full((16,), N, jnp.int32)

    def fill(i, carry):
        sb0_v[pl.ds(i * 16, 16)] = zs
        sb1_v[pl.ds(i * 16, 16)] = zs
        db0_v[pl.ds(i * 16, 16)] = ns
        db1_v[pl.ds(i * 16, 16)] = ns
        return carry

    lax.fori_loop(0, BCAP // 16, fill, 0)

    lanes = lax.iota(jnp.int32, 16)

    def body(i, cur):
        c0, c1 = cur
        s_v = src_v[pl.ds(i * 16, 16)]
        d_v = dst_v[pl.ds(i * 16, 16)]
        m0 = d_v < NHALF
        m1 = jnp.logical_not(m0)
        idx0 = jnp.minimum(lanes * LCAP + c0, lanes * LCAP + LCAP - 1)
        idx1 = jnp.minimum(lanes * LCAP + c1, lanes * LCAP + LCAP - 1)
        plsc.store_scatter(sb0_v, [idx0], s_v, mask=m0)
        plsc.store_scatter(db0_v, [idx0], d_v, mask=m0)
        plsc.store_scatter(sb1_v, [idx1], s_v, mask=m1)
        plsc.store_scatter(db1_v, [idx1], d_v, mask=m1)
        c0 = c0 + m0.astype(jnp.int32)
        c1 = c1 + m1.astype(jnp.int32)
        return c0, c1

    zc = jnp.zeros((16,), jnp.int32)
    lax.fori_loop(0, ET // 16, body, (zc, zc))

    @pl.when(cid == 0)
    def _():
        pltpu.sync_copy(sb0_v, sb_hbm.at[sid, 0])
        pltpu.sync_copy(sb1_v, sb_hbm.at[sid, 1])
        pltpu.sync_copy(db0_v, db_hbm.at[sid, 0])
        pltpu.sync_copy(db1_v, db_hbm.at[sid, 1])


@jax.jit
def _part(srcf, dstf):
    return pl.kernel(
        _part_body,
        out_type=(jax.ShapeDtypeStruct((NS, 2, BCAP), jnp.int32),
                  jax.ShapeDtypeStruct((NS, 2, BCAP), jnp.int32)),
        mesh=_mesh,
        compiler_params=pltpu.CompilerParams(needs_layout_passes=False),
        scratch_types=[
            pltpu.VMEM((ET,), jnp.int32),
            pltpu.VMEM((ET,), jnp.int32),
            pltpu.VMEM((BCAP,), jnp.int32),
            pltpu.VMEM((BCAP,), jnp.int32),
            pltpu.VMEM((BCAP,), jnp.int32),
            pltpu.VMEM((BCAP,), jnp.int32),
        ],
    )(srcf, dstf)


def _agg_body(h_hbm, srcr_hbm, dstr_hbm, z_hbm, out_hbm,
              src_v, dst_v, buf0, h_sp, acc, sem0):
    cid = lax.axis_index("c")
    sid = lax.axis_index("s")
    @pl.when(sid < NS - 1)
    def _():
        pltpu.sync_copy(h_hbm.at[pl.ds(sid * HT, HT)],
                        h_sp.at[pl.ds(sid * HT, HT)])
        pltpu.sync_copy(z_hbm.at[pl.ds(0, RPA)],
                        acc.at[pl.ds(sid * RPA, RPA)])

    @pl.when(sid == NS - 1)
    def _():
        pltpu.sync_copy(h_hbm.at[pl.ds((NS - 1) * HT, HTL)],
                        h_sp.at[pl.ds((NS - 1) * HT, HTL)])
        pltpu.sync_copy(z_hbm, acc.at[pl.ds((NS - 1) * RPA, RPAL)])

    plsc.subcore_barrier()

    lanes = lax.iota(jnp.int32, 16)
    lo = cid * NHALF

    def window(w, carry):
        pltpu.sync_copy(srcr_hbm.at[sid, cid, pl.ds(w * WIN, WIN)], src_v)
        pltpu.sync_copy(dstr_hbm.at[sid, cid, pl.ds(w * WIN, WIN)], dst_v)
        for r in range(WIN):
            for c2 in range(CA // 16):
                v = dst_v[r, pl.ds(c2 * 16, 16)]
                loc = v - lo
                bad = (loc < 0) | (loc >= NHALF)
                dst_v[r, pl.ds(c2 * 16, 16)] = jnp.where(
                    bad, NHALF + (lanes & (TRR - 1)), loc)

        def body(t, c3):
            pltpu.async_copy(h_sp.at[src_v.at[t]], buf0, sem0).wait()
            pltpu.sync_copy(buf0, acc.at[dst_v.at[t]], add=True)
            return c3

        lax.fori_loop(0, WIN, body, 0)
        return carry

    lax.fori_loop(0, KB // WIN, window, 0)
    plsc.subcore_barrier()

    @pl.when(sid < NS - 1)
    def _():
        pltpu.sync_copy(acc.at[pl.ds(sid * RPA, RPA)],
                        out_hbm.at[cid, pl.ds(sid * RPA, RPA)])

    @pl.when(sid == NS - 1)
    def _():
        pltpu.sync_copy(acc.at[pl.ds((NS - 1) * RPA, RPAL)],
                        out_hbm.at[cid, pl.ds((NS - 1) * RPA, RPAL)])


@jax.jit
def _agg(h, srcr, dstr, zrows):
    return pl.kernel(
        _agg_body,
        out_type=jax.ShapeDtypeStruct((NC, ACCR, D), jnp.float32),
        mesh=_mesh,
        scratch_types=[
            pltpu.VMEM((WIN, CA), jnp.int32),
            pltpu.VMEM((WIN, CA), jnp.int32),
            pltpu.VMEM((CA, D), jnp.float32),
            pltpu.VMEM_SHARED((N, D), jnp.float32),
            pltpu.VMEM_SHARED((ACCR, D), jnp.float32),
            pltpu.SemaphoreType.DMA,
        ],
    )(h, srcr, dstr, zrows)


BR = 1024


def _mlp_body(h_ref, p_ref, w1_ref, b1_ref, w2_ref, b2_ref, o_ref):
    z = h_ref[...] + p_ref[...]
    y = jnp.dot(z, w1_ref[...], preferred_element_type=jnp.float32)
    y = jnp.maximum(y + b1_ref[...], 0.0)
    y = jnp.dot(y, w2_ref[...], preferred_element_type=jnp.float32)
    o_ref[...] = jnp.maximum(y + b2_ref[...], 0.0)


@jax.jit
def _mlp(h, p, w1, b1, w2, b2):
    return pl.pallas_call(
        _mlp_body,
        grid=(NP // BR,),
        in_specs=[
            pl.BlockSpec((BR, D), lambda i: (i, 0)),
            pl.BlockSpec((BR, D), lambda i: (i, 0)),
            pl.BlockSpec((D, H), lambda i: (0, 0)),
            pl.BlockSpec((1, H), lambda i: (0, 0)),
            pl.BlockSpec((H, H), lambda i: (0, 0)),
            pl.BlockSpec((1, H), lambda i: (0, 0)),
        ],
        out_specs=pl.BlockSpec((BR, D), lambda i: (i, 0)),
        out_shape=jax.ShapeDtypeStruct((NP, D), jnp.float32),
    )(h, p, w1, b1, w2, b2)


def _mlp3_body(h_ref, p_ref, b_ref, w1_ref, b1_ref, w2_ref, b2_ref,
               wm1_ref, bm1_ref, wm2_ref, bm2_ref, o_ref,
               pool_ref, cnt_ref):
    i = pl.program_id(0)
    z = h_ref[...] + p_ref[...]
    y = jnp.dot(z, w1_ref[...], preferred_element_type=jnp.float32)
    y = jnp.maximum(y + b1_ref[...], 0.0)
    y = jnp.dot(y, w2_ref[...], preferred_element_type=jnp.float32)
    y = jnp.maximum(y + b2_ref[...], 0.0)

    gids = lax.broadcasted_iota(jnp.int32, (BR, G), 1)
    oh = (b_ref[...] == gids).astype(jnp.float32)
    pool = lax.dot_general(oh, y, (((0,), (0,)), ((), ())),
                           preferred_element_type=jnp.float32)
    cnt = lax.dot_general(oh, jnp.ones((BR, D), jnp.float32),
                          (((0,), (0,)), ((), ())),
                          preferred_element_type=jnp.float32)

    @pl.when(i == 0)
    def _():
        pool_ref[...] = pool
        cnt_ref[...] = cnt

    @pl.when(i > 0)
    def _():
        pool_ref[...] += pool
        cnt_ref[...] += cnt

    @pl.when(i == NP // BR - 1)
    def _():
        g = pool_ref[...] / jnp.maximum(cnt_ref[...], 1.0)
        u = jnp.dot(g, wm1_ref[...], preferred_element_type=jnp.float32)
        u = jnp.maximum(u + bm1_ref[...], 0.0)
        o = jnp.dot(u, wm2_ref[...], preferred_element_type=jnp.float32)
        o = o + bm2_ref[...]
        m = jnp.max(o, axis=1, keepdims=True)
        e = jnp.exp(o - m)
        o_ref[...] = e / jnp.sum(e, axis=1, keepdims=True)


@jax.jit
def _mlp3(h, p, batch2d, w1, b1, w2, b2, wm1, bm1, wm2p, bm2p):
    return pl.pallas_call(
        _mlp3_body,
        grid=(NP // BR,),
        in_specs=[
            pl.BlockSpec((BR, D), lambda i: (i, 0)),
            pl.BlockSpec((BR, D), lambda i: (i, 0)),
            pl.BlockSpec((BR, 1), lambda i: (i, 0)),
            pl.BlockSpec((D, H), lambda i: (0, 0)),
            pl.BlockSpec((1, H), lambda i: (0, 0)),
            pl.BlockSpec((H, H), lambda i: (0, 0)),
            pl.BlockSpec((1, H), lambda i: (0, 0)),
            pl.BlockSpec((H, G), lambda i: (0, 0)),
            pl.BlockSpec((1, G), lambda i: (0, 0)),
            pl.BlockSpec((G, H), lambda i: (0, 0)),
            pl.BlockSpec((1, H), lambda i: (0, 0)),
        ],
        out_specs=pl.BlockSpec((G, H), lambda i: (0, 0)),
        out_shape=jax.ShapeDtypeStruct((G, H), jnp.float32),
        scratch_shapes=[
            pltpu.VMEM((G, D), jnp.float32),
            pltpu.VMEM((G, D), jnp.float32),
        ],
    )(h, p, batch2d, w1, b1, w2, b2, wm1, bm1, wm2p, bm2p)


def _assemble(p):
    full = jnp.concatenate([p[0, :NHALF], p[1, :NHALF]], axis=0)
    return jnp.pad(full, ((0, NP - N), (0, 0)))


def kernel(x, edge_index, batch, emb,
           W1_0, b1_0, W2_0, b2_0,
           W1_1, b1_1, W2_1, b2_1,
           W1_2, b1_2, W2_2, b2_2,
           Wm1, bm1, Wm2, bm2):
    x = x.astype(jnp.int32)
    src = edge_index[0].astype(jnp.int32)
    dst = edge_index[1].astype(jnp.int32)
    batch = batch.astype(jnp.int32)

    xp = jnp.pad(x, (0, NP - N))
    xr = xp.reshape(NC, NS, KE, CE)

    pad = EP - E
    srcf = jnp.pad(src, (0, pad)).reshape(NS, ET)
    dstf = jnp.pad(dst, (0, pad), constant_values=N).reshape(NS, ET)

    zrows = jnp.zeros((RPAL, D), jnp.float32)
    batch2d = jnp.pad(batch, (0, NP - N), constant_values=G).reshape(NP, 1)

    b1_0r, b2_0r = b1_0.reshape(1, H), b2_0.reshape(1, H)
    b1_1r, b2_1r = b1_1.reshape(1, H), b2_1.reshape(1, H)
    b1_2r, b2_2r = b1_2.reshape(1, H), b2_2.reshape(1, H)
    bm1r = bm1.reshape(1, G)
    wm2p = jnp.pad(Wm2, ((0, 0), (0, H - 2)))
    bm2p = jnp.concatenate([bm2, jnp.full((H - 2,), -1e30, jnp.float32)])
    bm2p = bm2p.reshape(1, H)

    h = _embed(emb, xr)
    sb, db = _part(srcf, dstf)
    srcr = sb.reshape(NS, NC, KB, CA)
    dstr = db.reshape(NS, NC, KB, CA)

    p = _assemble(_agg(h, srcr, dstr, zrows))
    h = _mlp(h, p, W1_0, b1_0r, W2_0, b2_0r)
    p = _assemble(_agg(h, srcr, dstr, zrows))
    h = _mlp(h, p, W1_1, b1_1r, W2_1, b2_1r)
    p = _assemble(_agg(h, srcr, dstr, zrows))
    out = _mlp3(h, p, batch2d, W1_2, b1_2r, W2_2, b2_2r,
                Wm1, bm1r, wm2p, bm2p)
    return out[:, :2]


# speedup vs baseline: 1.1829x; 1.1829x over previous
import jax
import jax.numpy as jnp
from jax import lax
from jax.experimental import pallas as pl
from jax.experimental.pallas import tpu as pltpu
from jax.experimental.pallas import tpu_sc as plsc

N = 10000
E = 320000
V = 100000
D = 128
H = 128
G = 64

NC = 2
NS = 16
NW = NC * NS

NP = 10240
CE = 80
KE = NP // NW // CE

NHALF = N // 2
TRR = 8
ACCR = NHALF + TRR
RPA = 312
RPAL = ACCR - (NS - 1) * RPA
HT = 624
HTL = N - (NS - 1) * HT

CA = 32
ET = 20480
WIN = 8
EP = ET * NS
LCAP = 800
BCAP = 16 * LCAP
KB = BCAP // CA

_mesh = plsc.VectorSubcoreMesh(
    core_axis_name="c", subcore_axis_name="s", num_cores=NC, num_subcores=NS)


def _embed_body(emb_hbm, xr_hbm, out_hbm, idx_v, buf_v, sem):
    cid = lax.axis_index("c")
    sid = lax.axis_index("s")
    wid = cid * NS + sid
    pltpu.sync_copy(xr_hbm.at[cid, sid], idx_v)
    base = wid * (KE * CE)

    def body(j, carry):
        pltpu.async_copy(emb_hbm.at[idx_v.at[j]], buf_v, sem).wait()
        pltpu.sync_copy(buf_v, out_hbm.at[pl.ds(base + j * CE, CE)])
        return carry

    lax.fori_loop(0, KE, body, 0)


@jax.jit
def _embed(emb, xr):
    return pl.kernel(
        _embed_body,
        out_type=jax.ShapeDtypeStruct((NP, D), jnp.float32),
        mesh=_mesh,
        scratch_types=[
            pltpu.VMEM((KE, CE), jnp.int32),
            pltpu.VMEM((CE, D), jnp.float32),
            pltpu.SemaphoreType.DMA,
        ],
    )(emb, xr)


def _part_body(srcf_hbm, dstf_hbm, sb_hbm, db_hbm, src_v, dst_v,
               sb0_v, sb1_v, db0_v, db1_v):
    cid = lax.axis_index("c")
    sid = lax.axis_index("s")

    pltpu.sync_copy(srcf_hbm.at[sid], src_v)
    pltpu.sync_copy(dstf_hbm.at[sid], dst_v)

    lanes = lax.iota(jnp.int32, 16)
    zs = jnp.zeros((16,), jnp.int32)
    ns = NHALF + (lanes & (TRR - 1))

    def fill(i, carry):
        sb0_v[pl.ds(i * 16, 16)] = zs
        sb1_v[pl.ds(i * 16, 16)] = zs
        db0_v[pl.ds(i * 16, 16)] = ns
        db1_v[pl.ds(i * 16, 16)] = ns
        return carry

    lax.fori_loop(0, BCAP // 16, fill, 0)

    def body(i, cur):
        c0, c1 = cur
        s_v = src_v[pl.ds(i * 16, 16)]
        d_v = dst_v[pl.ds(i * 16, 16)]
        m0 = d_v < NHALF
        m1 = jnp.logical_not(m0)
        idx0 = jnp.minimum(lanes * LCAP + c0, lanes * LCAP + LCAP - 1)
        idx1 = jnp.minimum(lanes * LCAP + c1, lanes * LCAP + LCAP - 1)
        plsc.store_scatter(sb0_v, [idx0], s_v, mask=m0)
        plsc.store_scatter(db0_v, [idx0], d_v, mask=m0)
        plsc.store_scatter(sb1_v, [idx1], s_v, mask=m1)
        plsc.store_scatter(db1_v, [idx1], d_v - NHALF, mask=m1)
        c0 = c0 + m0.astype(jnp.int32)
        c1 = c1 + m1.astype(jnp.int32)
        return c0, c1

    zc = jnp.zeros((16,), jnp.int32)
    lax.fori_loop(0, ET // 16, body, (zc, zc))

    @pl.when(cid == 0)
    def _():
        pltpu.sync_copy(sb0_v, sb_hbm.at[sid, 0])
        pltpu.sync_copy(sb1_v, sb_hbm.at[sid, 1])
        pltpu.sync_copy(db0_v, db_hbm.at[sid, 0])
        pltpu.sync_copy(db1_v, db_hbm.at[sid, 1])


@jax.jit
def _part(srcf, dstf):
    return pl.kernel(
        _part_body,
        out_type=(jax.ShapeDtypeStruct((NS, 2, BCAP), jnp.int32),
                  jax.ShapeDtypeStruct((NS, 2, BCAP), jnp.int32)),
        mesh=_mesh,
        compiler_params=pltpu.CompilerParams(needs_layout_passes=False),
        scratch_types=[
            pltpu.VMEM((ET,), jnp.int32),
            pltpu.VMEM((ET,), jnp.int32),
            pltpu.VMEM((BCAP,), jnp.int32),
            pltpu.VMEM((BCAP,), jnp.int32),
            pltpu.VMEM((BCAP,), jnp.int32),
            pltpu.VMEM((BCAP,), jnp.int32),
        ],
    )(srcf, dstf)


def _agg_body(h_hbm, srcr_hbm, dstr_hbm, z_hbm, out_hbm,
              src_v, dst_v, buf0, buf1, h_sp, acc, sem0, sem1):
    cid = lax.axis_index("c")
    sid = lax.axis_index("s")
    @pl.when(sid < NS - 1)
    def _():
        pltpu.sync_copy(h_hbm.at[pl.ds(sid * HT, HT)],
                        h_sp.at[pl.ds(sid * HT, HT)])
        pltpu.sync_copy(z_hbm.at[pl.ds(0, RPA)],
                        acc.at[pl.ds(sid * RPA, RPA)])

    @pl.when(sid == NS - 1)
    def _():
        pltpu.sync_copy(h_hbm.at[pl.ds((NS - 1) * HT, HTL)],
                        h_sp.at[pl.ds((NS - 1) * HT, HTL)])
        pltpu.sync_copy(z_hbm, acc.at[pl.ds((NS - 1) * RPA, RPAL)])

    plsc.subcore_barrier()

    def window(w, carry):
        pltpu.sync_copy(srcr_hbm.at[sid, cid, pl.ds(w * WIN, WIN)], src_v)
        pltpu.sync_copy(dstr_hbm.at[sid, cid, pl.ds(w * WIN, WIN)], dst_v)
        pltpu.async_copy(h_sp.at[src_v.at[0]], buf0, sem0)

        def body(t, c3):
            j0 = 2 * t
            pltpu.async_copy(h_sp.at[src_v.at[j0 + 1]], buf1, sem1)
            pltpu.make_async_copy(h_sp.at[src_v.at[j0]], buf0, sem0).wait()
            pltpu.sync_copy(buf0, acc.at[dst_v.at[j0]], add=True)

            @pl.when(t < WIN // 2 - 1)
            def _():
                pltpu.async_copy(h_sp.at[src_v.at[j0 + 2]], buf0, sem0)

            pltpu.make_async_copy(h_sp.at[src_v.at[j0 + 1]], buf1,
                                  sem1).wait()
            pltpu.sync_copy(buf1, acc.at[dst_v.at[j0 + 1]], add=True)
            return c3

        lax.fori_loop(0, WIN // 2, body, 0)
        return carry

    lax.fori_loop(0, KB // WIN, window, 0)
    plsc.subcore_barrier()

    @pl.when(sid < NS - 1)
    def _():
        pltpu.sync_copy(acc.at[pl.ds(sid * RPA, RPA)],
                        out_hbm.at[cid, pl.ds(sid * RPA, RPA)])

    @pl.when(sid == NS - 1)
    def _():
        pltpu.sync_copy(acc.at[pl.ds((NS - 1) * RPA, RPAL)],
                        out_hbm.at[cid, pl.ds((NS - 1) * RPA, RPAL)])


@jax.jit
def _agg(h, srcr, dstr, zrows):
    return pl.kernel(
        _agg_body,
        out_type=jax.ShapeDtypeStruct((NC, ACCR, D), jnp.float32),
        mesh=_mesh,
        scratch_types=[
            pltpu.VMEM((WIN, CA), jnp.int32),
            pltpu.VMEM((WIN, CA), jnp.int32),
            pltpu.VMEM((CA, D), jnp.float32),
            pltpu.VMEM((CA, D), jnp.float32),
            pltpu.VMEM_SHARED((N, D), jnp.float32),
            pltpu.VMEM_SHARED((ACCR, D), jnp.float32),
            pltpu.SemaphoreType.DMA,
            pltpu.SemaphoreType.DMA,
        ],
    )(h, srcr, dstr, zrows)


BR = 1024


def _mlp_body(h_ref, p_ref, w1_ref, b1_ref, w2_ref, b2_ref, o_ref):
    z = h_ref[...] + p_ref[...]
    y = jnp.dot(z, w1_ref[...], preferred_element_type=jnp.float32)
    y = jnp.maximum(y + b1_ref[...], 0.0)
    y = jnp.dot(y, w2_ref[...], preferred_element_type=jnp.float32)
    o_ref[...] = jnp.maximum(y + b2_ref[...], 0.0)


@jax.jit
def _mlp(h, p, w1, b1, w2, b2):
    return pl.pallas_call(
        _mlp_body,
        grid=(NP // BR,),
        in_specs=[
            pl.BlockSpec((BR, D), lambda i: (i, 0)),
            pl.BlockSpec((BR, D), lambda i: (i, 0)),
            pl.BlockSpec((D, H), lambda i: (0, 0)),
            pl.BlockSpec((1, H), lambda i: (0, 0)),
            pl.BlockSpec((H, H), lambda i: (0, 0)),
            pl.BlockSpec((1, H), lambda i: (0, 0)),
        ],
        out_specs=pl.BlockSpec((BR, D), lambda i: (i, 0)),
        out_shape=jax.ShapeDtypeStruct((NP, D), jnp.float32),
    )(h, p, w1, b1, w2, b2)


def _mlp3_body(h_ref, p_ref, b_ref, w1_ref, b1_ref, w2_ref, b2_ref,
               wm1_ref, bm1_ref, wm2_ref, bm2_ref, o_ref,
               pool_ref, cnt_ref):
    i = pl.program_id(0)
    z = h_ref[...] + p_ref[...]
    y = jnp.dot(z, w1_ref[...], preferred_element_type=jnp.float32)
    y = jnp.maximum(y + b1_ref[...], 0.0)
    y = jnp.dot(y, w2_ref[...], preferred_element_type=jnp.float32)
    y = jnp.maximum(y + b2_ref[...], 0.0)

    gids = lax.broadcasted_iota(jnp.int32, (BR, G), 1)
    oh = (b_ref[...] == gids).astype(jnp.float32)
    pool = lax.dot_general(oh, y, (((0,), (0,)), ((), ())),
                           preferred_element_type=jnp.float32)
    cnt = lax.dot_general(oh, jnp.ones((BR, D), jnp.float32),
                          (((0,), (0,)), ((), ())),
                          preferred_element_type=jnp.float32)

    @pl.when(i == 0)
    def _():
        pool_ref[...] = pool
        cnt_ref[...] = cnt

    @pl.when(i > 0)
    def _():
        pool_ref[...] += pool
        cnt_ref[...] += cnt

    @pl.when(i == NP // BR - 1)
    def _():
        g = pool_ref[...] / jnp.maximum(cnt_ref[...], 1.0)
        u = jnp.dot(g, wm1_ref[...], preferred_element_type=jnp.float32)
        u = jnp.maximum(u + bm1_ref[...], 0.0)
        o = jnp.dot(u, wm2_ref[...], preferred_element_type=jnp.float32)
        o = o + bm2_ref[...]
        m = jnp.max(o, axis=1, keepdims=True)
        e = jnp.exp(o - m)
        o_ref[...] = e / jnp.sum(e, axis=1, keepdims=True)


@jax.jit
def _mlp3(h, p, batch2d, w1, b1, w2, b2, wm1, bm1, wm2p, bm2p):
    return pl.pallas_call(
        _mlp3_body,
        grid=(NP // BR,),
        in_specs=[
            pl.BlockSpec((BR, D), lambda i: (i, 0)),
            pl.BlockSpec((BR, D), lambda i: (i, 0)),
            pl.BlockSpec((BR, 1), lambda i: (i, 0)),
            pl.BlockSpec((D, H), lambda i: (0, 0)),
            pl.BlockSpec((1, H), lambda i: (0, 0)),
            pl.BlockSpec((H, H), lambda i: (0, 0)),
            pl.BlockSpec((1, H), lambda i: (0, 0)),
            pl.BlockSpec((H, G), lambda i: (0, 0)),
            pl.BlockSpec((1, G), lambda i: (0, 0)),
            pl.BlockSpec((G, H), lambda i: (0, 0)),
            pl.BlockSpec((1, H), lambda i: (0, 0)),
        ],
        out_specs=pl.BlockSpec((G, H), lambda i: (0, 0)),
        out_shape=jax.ShapeDtypeStruct((G, H), jnp.float32),
        scratch_shapes=[
            pltpu.VMEM((G, D), jnp.float32),
            pltpu.VMEM((G, D), jnp.float32),
        ],
    )(h, p, batch2d, w1, b1, w2, b2, wm1, bm1, wm2p, bm2p)


def _assemble(p):
    full = jnp.concatenate([p[0, :NHALF], p[1, :NHALF]], axis=0)
    return jnp.pad(full, ((0, NP - N), (0, 0)))


def kernel(x, edge_index, batch, emb,
           W1_0, b1_0, W2_0, b2_0,
           W1_1, b1_1, W2_1, b2_1,
           W1_2, b1_2, W2_2, b2_2,
           Wm1, bm1, Wm2, bm2):
    x = x.astype(jnp.int32)
    src = edge_index[0].astype(jnp.int32)
    dst = edge_index[1].astype(jnp.int32)
    batch = batch.astype(jnp.int32)

    xp = jnp.pad(x, (0, NP - N))
    xr = xp.reshape(NC, NS, KE, CE)

    pad = EP - E
    srcf = jnp.pad(src, (0, pad)).reshape(NS, ET)
    dst_fill = N + jnp.arange(pad, dtype=jnp.int32) % TRR
    dstf = jnp.concatenate([dst, dst_fill]).reshape(NS, ET)

    zrows = jnp.zeros((RPAL, D), jnp.float32)
    batch2d = jnp.pad(batch, (0, NP - N), constant_values=G).reshape(NP, 1)

    b1_0r, b2_0r = b1_0.reshape(1, H), b2_0.reshape(1, H)
    b1_1r, b2_1r = b1_1.reshape(1, H), b2_1.reshape(1, H)
    b1_2r, b2_2r = b1_2.reshape(1, H), b2_2.reshape(1, H)
    bm1r = bm1.reshape(1, G)
    wm2p = jnp.pad(Wm2, ((0, 0), (0, H - 2)))
    bm2p = jnp.concatenate([bm2, jnp.full((H - 2,), -1e30, jnp.float32)])
    bm2p = bm2p.reshape(1, H)

    h = _embed(emb, xr)
    sb, db = _part(srcf, dstf)
    srcr = sb.reshape(NS, NC, KB, CA)
    dstr = db.reshape(NS, NC, KB, CA)

    p = _assemble(_agg(h, srcr, dstr, zrows))
    h = _mlp(h, p, W1_0, b1_0r, W2_0, b2_0r)
    p = _assemble(_agg(h, srcr, dstr, zrows))
    h = _mlp(h, p, W1_1, b1_1r, W2_1, b2_1r)
    p = _assemble(_agg(h, srcr, dstr, zrows))
    out = _mlp3(h, p, batch2d, W1_2, b1_2r, W2_2, b2_2r,
                Wm1, bm1r, wm2p, bm2p)
    return out[:, :2]
